# baseline (speedup 1.0000x reference)
import functools

import jax
import jax.numpy as jnp
from jax import lax
from jax.experimental import pallas as pl
from jax.experimental.pallas import tpu as pltpu
from jax.experimental.pallas import tpu_sc as plsc

NUM_ITEM = 100000
EMBED_DIM = 128
BATCH = 16384
N_COLS = 10

_info = plsc.get_sparse_core_info()
_NC, _NS = _info.num_cores, _info.num_subcores
_NW = _NC * _NS
_CHUNK = 128
_B_PER_W = BATCH // _NW
_NCH = _B_PER_W // _CHUNK

_mesh = plsc.VectorSubcoreMesh(core_axis_name="c", subcore_axis_name="s")


@functools.partial(
    pl.kernel,
    mesh=_mesh,
    out_type=jax.ShapeDtypeStruct((BATCH, EMBED_DIM), jnp.float32),
    scratch_types=[
        pltpu.VMEM((_B_PER_W * N_COLS + 16,), jnp.int32),
        pltpu.VMEM((_NCH, _CHUNK), jnp.int32),
        pltpu.VMEM((_NCH, _CHUNK, EMBED_DIM), jnp.float32),
    ]
    + [pltpu.SemaphoreType.DMA] * (2 * _NCH),
)
def _gather_kernel(fea_hbm, table_hbm, out_hbm, fea_v, idx_v, rows_v, *sems):
    gsems, wsems = sems[:_NCH], sems[_NCH:]
    wid = lax.axis_index("s") * _NC + lax.axis_index("c")
    base = wid * _NCH
    rbase = wid * _B_PER_W
    lane = lax.iota(jnp.int32, 16)
    pltpu.sync_copy(
        fea_hbm.at[pl.ds(rbase * N_COLS, _B_PER_W * N_COLS)],
        fea_v.at[pl.ds(0, _B_PER_W * N_COLS)],
    )
    for j in range(_NCH):
        row = idx_v.at[j]

        def _extract(k, carry, j=j, row=row):
            fbase = j * _CHUNK * N_COLS + k * (16 * N_COLS)
            out = jnp.zeros((16,), jnp.int32)
            for u in range(16):
                v = fea_v[pl.ds(fbase + u * N_COLS, 16)]
                out = jnp.where(lane == u, v[0], out)
            row[pl.ds(k * 16, 16)] = out
            return carry

        lax.fori_loop(0, _CHUNK // 16, _extract, 0)
    gathers = [
        pltpu.async_copy(table_hbm.at[idx_v.at[j]], rows_v.at[j], gsems[j])
        for j in range(_NCH)
    ]
    writes = []
    for j in range(_NCH):
        gathers[j].wait()
        writes.append(
            pltpu.async_copy(
                rows_v.at[j],
                out_hbm.at[pl.ds((base + j) * _CHUNK, _CHUNK)],
                wsems[j],
            )
        )
    for w in writes:
        w.wait()


def kernel(item_fea, embedding_itemId):
    fea_flat = item_fea.astype(jnp.int32).reshape(BATCH * N_COLS)
    return _gather_kernel(fea_flat, embedding_itemId)


# speedup vs baseline: 1.5275x; 1.5275x over previous
import functools

import jax
import jax.numpy as jnp
from jax import lax
from jax.experimental import pallas as pl
from jax.experimental.pallas import tpu as pltpu
from jax.experimental.pallas import tpu_sc as plsc

NUM_ITEM = 100000
EMBED_DIM = 128
BATCH = 16384

_info = plsc.get_sparse_core_info()
_NC, _NS = _info.num_cores, _info.num_subcores
_NW = _NC * _NS
_CHUNK = 128
_B_PER_W = BATCH // _NW
_NCH = _B_PER_W // _CHUNK

_mesh = plsc.VectorSubcoreMesh(core_axis_name="c", subcore_axis_name="s")


@functools.partial(
    pl.kernel,
    mesh=_mesh,
    out_type=jax.ShapeDtypeStruct((BATCH, EMBED_DIM), jnp.float32),
    scratch_types=[
        pltpu.VMEM((_NCH, _CHUNK), jnp.int32),
        pltpu.VMEM((_NCH, _CHUNK, EMBED_DIM), jnp.float32),
    ]
    + [pltpu.SemaphoreType.DMA] * (3 * _NCH),
)
def _gather_kernel(idx_hbm, table_hbm, out_hbm, idx_v, rows_v, *sems):
    isems = sems[:_NCH]
    gsems = sems[_NCH : 2 * _NCH]
    wsems = sems[2 * _NCH :]
    wid = lax.axis_index("s") * _NC + lax.axis_index("c")
    base = wid * _NCH
    idx_loads = [
        pltpu.async_copy(idx_hbm.at[pl.ds(base + j, 1)], idx_v.at[pl.ds(j, 1)],
                         isems[j])
        for j in range(_NCH)
    ]
    gathers = []
    for j in range(_NCH):
        idx_loads[j].wait()
        gathers.append(
            pltpu.async_copy(table_hbm.at[idx_v.at[j]], rows_v.at[j], gsems[j])
        )
    writes = []
    for j in range(_NCH):
        gathers[j].wait()
        writes.append(
            pltpu.async_copy(
                rows_v.at[j],
                out_hbm.at[pl.ds((base + j) * _CHUNK, _CHUNK)],
                wsems[j],
            )
        )
    for w in writes:
        w.wait()


def kernel(item_fea, embedding_itemId):
    idx = item_fea[:, 0].astype(jnp.int32).reshape(BATCH // _CHUNK, _CHUNK)
    return _gather_kernel(idx, embedding_itemId)


# speedup vs baseline: 1.5375x; 1.0065x over previous
import functools

import jax
import jax.numpy as jnp
from jax import lax
from jax.experimental import pallas as pl
from jax.experimental.pallas import tpu as pltpu
from jax.experimental.pallas import tpu_sc as plsc

NUM_ITEM = 100000
EMBED_DIM = 128
BATCH = 16384

_info = plsc.get_sparse_core_info()
_NC, _NS = _info.num_cores, _info.num_subcores
_NW = _NC * _NS
_CHUNK = 128
_B_PER_W = BATCH // _NW
_NCH = _B_PER_W // _CHUNK

_mesh = plsc.VectorSubcoreMesh(core_axis_name="c", subcore_axis_name="s")


@functools.partial(
    pl.kernel,
    mesh=_mesh,
    out_type=jax.ShapeDtypeStruct((BATCH, EMBED_DIM), jnp.float32),
    scratch_types=[
        pltpu.VMEM((_NCH, _CHUNK), jnp.int32),
        pltpu.VMEM((_NCH, _CHUNK, EMBED_DIM), jnp.float32),
    ]
    + [pltpu.SemaphoreType.DMA] * (2 * _NCH),
)
def _gather_kernel(idx_hbm, table_hbm, out_hbm, idx_v, rows_v, *sems):
    gsems, wsems = sems[:_NCH], sems[_NCH:]
    wid = lax.axis_index("s") * _NC + lax.axis_index("c")
    base = wid * _NCH
    pltpu.sync_copy(idx_hbm.at[pl.ds(base, _NCH)], idx_v)
    gathers = [
        pltpu.async_copy(table_hbm.at[idx_v.at[j]], rows_v.at[j], gsems[j])
        for j in range(_NCH)
    ]
    writes = []
    for j in range(_NCH):
        gathers[j].wait()
        writes.append(
            pltpu.async_copy(
                rows_v.at[j],
                out_hbm.at[pl.ds((base + j) * _CHUNK, _CHUNK)],
                wsems[j],
            )
        )
    for w in writes:
        w.wait()


def kernel(item_fea, embedding_itemId):
    idx = item_fea[:, 0].astype(jnp.int32).reshape(BATCH // _CHUNK, _CHUNK)
    return _gather_kernel(idx, embedding_itemId)
